# Initial kernel scaffold; baseline (speedup 1.0000x reference)
import jax
import jax.numpy as jnp
from jax.experimental import pallas as pl


def kernel(x, W_gate):
    raise NotImplementedError("write your pallas kernel here")


# speedup vs baseline: 1.4606x; 1.4606x over previous
import functools

import jax
import jax.numpy as jnp
from jax import lax
from jax.experimental import pallas as pl
from jax.experimental.pallas import tpu as pltpu
from jax.experimental.pallas import tpu_sc as plsc

NUM_TOKENS = 16384
MODEL_DIM = 2048
NUM_EXPERTS = 64
LANES = 16
NUM_CORES = 2
NUM_SUBCORES = 16
NUM_WORKERS = NUM_CORES * NUM_SUBCORES
TOKENS_PER_WORKER = NUM_TOKENS // NUM_WORKERS
GROUPS_PER_WORKER = TOKENS_PER_WORKER // LANES


def _matmul_body(w_ref, x_ref, out_ref):
    out_ref[0] = lax.dot_general(
        w_ref[...], x_ref[...],
        dimension_numbers=(((1,), (1,)), ((), ())),
        preferred_element_type=jnp.float32,
        precision=lax.Precision.DEFAULT,
    )


def _gate_scores_blocked(x, w_gate):
    tb = TOKENS_PER_WORKER
    grid = NUM_TOKENS // tb
    return pl.pallas_call(
        _matmul_body,
        grid=(grid,),
        in_specs=[
            pl.BlockSpec((NUM_EXPERTS, MODEL_DIM), lambda i: (0, 0)),
            pl.BlockSpec((tb, MODEL_DIM), lambda i: (i, 0)),
        ],
        out_specs=pl.BlockSpec((1, NUM_EXPERTS, tb), lambda i: (i, 0, 0)),
        out_shape=jax.ShapeDtypeStruct(
            (grid, NUM_EXPERTS, tb), jnp.float32),
    )(w_gate, x)


def _sc_topk_body(scores_hbm, g1_hbm, g2_hbm, i1_hbm, i2_hbm,
                  sbuf, g1v, g2v, i1v, i2v):
    cid = lax.axis_index("c")
    sid = lax.axis_index("s")
    wid = sid * NUM_CORES + cid
    pltpu.sync_copy(scores_hbm.at[wid], sbuf)

    def group(t, carry):
        base = t * LANES
        m1 = jnp.full((LANES,), -jnp.inf, jnp.float32)
        m2 = jnp.full((LANES,), -jnp.inf, jnp.float32)
        i1 = jnp.zeros((LANES,), jnp.int32)
        i2 = jnp.zeros((LANES,), jnp.int32)
        for e in range(NUM_EXPERTS):
            v = sbuf[e, pl.ds(base, LANES)]
            ev = jnp.full((LANES,), e, jnp.int32)
            gt1 = v > m1
            gt2 = v > m2
            i2 = jnp.where(gt1, i1, jnp.where(gt2, ev, i2))
            m2 = jnp.where(gt1, m1, jnp.where(gt2, v, m2))
            i1 = jnp.where(gt1, ev, i1)
            m1 = jnp.where(gt1, v, m1)
        e2 = jnp.exp(m2 - m1)
        den = 1.0 + e2
        g1v[pl.ds(base, LANES)] = 1.0 / den
        g2v[pl.ds(base, LANES)] = e2 / den
        i1v[pl.ds(base, LANES)] = i1
        i2v[pl.ds(base, LANES)] = i2
        return carry

    lax.fori_loop(0, GROUPS_PER_WORKER, group, 0)

    out_slice = pl.ds(wid * TOKENS_PER_WORKER, TOKENS_PER_WORKER)
    pltpu.sync_copy(g1v, g1_hbm.at[out_slice])
    pltpu.sync_copy(g2v, g2_hbm.at[out_slice])
    pltpu.sync_copy(i1v, i1_hbm.at[out_slice])
    pltpu.sync_copy(i2v, i2_hbm.at[out_slice])


@functools.lru_cache(maxsize=1)
def _sc_topk():
    return pl.kernel(
        _sc_topk_body,
        out_type=(
            jax.ShapeDtypeStruct((NUM_TOKENS,), jnp.float32),
            jax.ShapeDtypeStruct((NUM_TOKENS,), jnp.float32),
            jax.ShapeDtypeStruct((NUM_TOKENS,), jnp.int32),
            jax.ShapeDtypeStruct((NUM_TOKENS,), jnp.int32),
        ),
        mesh=plsc.VectorSubcoreMesh(
            core_axis_name="c", subcore_axis_name="s",
            num_cores=NUM_CORES, num_subcores=NUM_SUBCORES),
        scratch_types=(
            pltpu.VMEM((NUM_EXPERTS, TOKENS_PER_WORKER), jnp.float32),
            pltpu.VMEM((TOKENS_PER_WORKER,), jnp.float32),
            pltpu.VMEM((TOKENS_PER_WORKER,), jnp.float32),
            pltpu.VMEM((TOKENS_PER_WORKER,), jnp.int32),
            pltpu.VMEM((TOKENS_PER_WORKER,), jnp.int32),
        ),
    )


def kernel(x, W_gate):
    scores = _gate_scores_blocked(x, W_gate)
    g1, g2, i1, i2 = _sc_topk()(scores)
    top_k_gates = jnp.stack([g1, g2], axis=-1)
    top_k_indices = jnp.stack([i1, i2], axis=-1)
    return top_k_gates, top_k_indices


# speedup vs baseline: 1.5969x; 1.0933x over previous
import functools

import jax
import jax.numpy as jnp
from jax import lax
from jax.experimental import pallas as pl
from jax.experimental.pallas import tpu as pltpu
from jax.experimental.pallas import tpu_sc as plsc

NUM_TOKENS = 16384
MODEL_DIM = 2048
NUM_EXPERTS = 64
LANES = 16
NUM_CORES = 2
NUM_SUBCORES = 16
NUM_WORKERS = NUM_CORES * NUM_SUBCORES
TOKENS_PER_WORKER = NUM_TOKENS // NUM_WORKERS
GROUPS_PER_WORKER = TOKENS_PER_WORKER // LANES


TOKEN_BLOCK = 2048
WORKERS_PER_BLOCK = TOKEN_BLOCK // TOKENS_PER_WORKER


def _matmul_body(w_ref, x_ref, out_ref):
    res = lax.dot_general(
        w_ref[...], x_ref[...],
        dimension_numbers=(((1,), (1,)), ((), ())),
        preferred_element_type=jnp.float32,
        precision=lax.Precision.DEFAULT,
    )
    for k in range(WORKERS_PER_BLOCK):
        out_ref[k] = res[:, k * TOKENS_PER_WORKER:(k + 1) * TOKENS_PER_WORKER]


def _gate_scores_blocked(x, w_gate):
    grid = NUM_TOKENS // TOKEN_BLOCK
    return pl.pallas_call(
        _matmul_body,
        grid=(grid,),
        in_specs=[
            pl.BlockSpec((NUM_EXPERTS, MODEL_DIM), lambda i: (0, 0)),
            pl.BlockSpec((TOKEN_BLOCK, MODEL_DIM), lambda i: (i, 0)),
        ],
        out_specs=pl.BlockSpec(
            (WORKERS_PER_BLOCK, NUM_EXPERTS, TOKENS_PER_WORKER),
            lambda i: (i, 0, 0)),
        out_shape=jax.ShapeDtypeStruct(
            (NUM_WORKERS, NUM_EXPERTS, TOKENS_PER_WORKER), jnp.float32),
    )(w_gate, x)


def _sc_topk_body(scores_hbm, g1_hbm, g2_hbm, i1_hbm, i2_hbm,
                  sbuf, g1v, g2v, i1v, i2v):
    cid = lax.axis_index("c")
    sid = lax.axis_index("s")
    wid = sid * NUM_CORES + cid
    pltpu.sync_copy(scores_hbm.at[wid], sbuf)

    def group(t, carry):
        base = t * LANES
        m1 = jnp.full((LANES,), -jnp.inf, jnp.float32)
        m2 = jnp.full((LANES,), -jnp.inf, jnp.float32)
        i1 = jnp.zeros((LANES,), jnp.int32)
        i2 = jnp.zeros((LANES,), jnp.int32)
        for e in range(NUM_EXPERTS):
            v = sbuf[e, pl.ds(base, LANES)]
            ev = jnp.full((LANES,), e, jnp.int32)
            gt1 = v > m1
            gt2 = v > m2
            i2 = jnp.where(gt1, i1, jnp.where(gt2, ev, i2))
            m2 = jnp.where(gt1, m1, jnp.where(gt2, v, m2))
            i1 = jnp.where(gt1, ev, i1)
            m1 = jnp.where(gt1, v, m1)
        e2 = jnp.exp(m2 - m1)
        den = 1.0 + e2
        g1v[pl.ds(base, LANES)] = 1.0 / den
        g2v[pl.ds(base, LANES)] = e2 / den
        i1v[pl.ds(base, LANES)] = i1
        i2v[pl.ds(base, LANES)] = i2
        return carry

    lax.fori_loop(0, GROUPS_PER_WORKER, group, 0)

    out_slice = pl.ds(wid * TOKENS_PER_WORKER, TOKENS_PER_WORKER)
    pltpu.sync_copy(g1v, g1_hbm.at[out_slice])
    pltpu.sync_copy(g2v, g2_hbm.at[out_slice])
    pltpu.sync_copy(i1v, i1_hbm.at[out_slice])
    pltpu.sync_copy(i2v, i2_hbm.at[out_slice])


@functools.lru_cache(maxsize=1)
def _sc_topk():
    return pl.kernel(
        _sc_topk_body,
        out_type=(
            jax.ShapeDtypeStruct((NUM_TOKENS,), jnp.float32),
            jax.ShapeDtypeStruct((NUM_TOKENS,), jnp.float32),
            jax.ShapeDtypeStruct((NUM_TOKENS,), jnp.int32),
            jax.ShapeDtypeStruct((NUM_TOKENS,), jnp.int32),
        ),
        mesh=plsc.VectorSubcoreMesh(
            core_axis_name="c", subcore_axis_name="s",
            num_cores=NUM_CORES, num_subcores=NUM_SUBCORES),
        scratch_types=(
            pltpu.VMEM((NUM_EXPERTS, TOKENS_PER_WORKER), jnp.float32),
            pltpu.VMEM((TOKENS_PER_WORKER,), jnp.float32),
            pltpu.VMEM((TOKENS_PER_WORKER,), jnp.float32),
            pltpu.VMEM((TOKENS_PER_WORKER,), jnp.int32),
            pltpu.VMEM((TOKENS_PER_WORKER,), jnp.int32),
        ),
    )


def kernel(x, W_gate):
    scores = _gate_scores_blocked(x, W_gate)
    g1, g2, i1, i2 = _sc_topk()(scores)
    top_k_gates = jnp.stack([g1, g2], axis=-1)
    top_k_indices = jnp.stack([i1, i2], axis=-1)
    return top_k_gates, top_k_indices
